# baseline (speedup 1.0000x reference)
import functools

import jax
import jax.numpy as jnp
from jax import lax
from jax.experimental import pallas as pl
from jax.experimental.pallas import tpu as pltpu
from jax.experimental.pallas import tpu_sc as plsc

N = 10000
E = 160000
NUM_FEAT = 128
DIM = 32
NGRAPH = 64
LW = 128

NW = 32
CHUNK = 128
CPW = 40
EP = NW * CPW * CHUNK
NPAD = 10112
RPT = NPAD // 16
GRP = 4
EPW = CPW * CHUNK

_SC_PARAMS = pltpu.CompilerParams(use_tc_tiling_on_sc=False)

_f32 = jnp.float32
_bf16 = jnp.bfloat16
_i32 = jnp.int32


def _sc_gather(table, idx2d):
    mesh = plsc.VectorSubcoreMesh(core_axis_name="c", subcore_axis_name="s")

    @functools.partial(
        pl.kernel, mesh=mesh, compiler_params=_SC_PARAMS,
        out_type=jax.ShapeDtypeStruct((EP, LW), _f32),
        scratch_types=[
            pltpu.VMEM((CPW, CHUNK), _i32),
            pltpu.VMEM((GRP * CHUNK, LW), _f32),
            pltpu.SemaphoreType.DMA,
        ],
    )
    def k(table_hbm, idx_hbm, out_hbm, idx_v, rows_v, sem):
        c = lax.axis_index("c")
        s = lax.axis_index("s")
        wid = s * 2 + c
        pltpu.sync_copy(idx_hbm.at[pl.ds(wid * CPW, CPW)], idx_v)

        def body(j, carry):
            cps = [
                pltpu.async_copy(
                    table_hbm.at[idx_v.at[j * GRP + b]],
                    rows_v.at[pl.ds(b * CHUNK, CHUNK)],
                    sem,
                )
                for b in range(GRP)
            ]
            for cp in cps:
                cp.wait()
            pltpu.sync_copy(
                rows_v,
                out_hbm.at[pl.ds(wid * EPW + j * GRP * CHUNK, GRP * CHUNK)],
            )
            return carry

        lax.fori_loop(0, CPW // GRP, body, 0)

    return k(table, idx2d)


def _sc_scatter(msg, idx2d, zeros_tab):
    mesh = plsc.VectorSubcoreMesh(core_axis_name="c", subcore_axis_name="s")
    ot = jax.ShapeDtypeStruct((NPAD, LW), _f32)

    @functools.partial(
        pl.kernel, mesh=mesh, compiler_params=_SC_PARAMS,
        out_type=(ot, ot),
        scratch_types=[
            pltpu.VMEM((CPW, CHUNK), _i32),
            pltpu.VMEM((GRP * CHUNK, 2 * DIM), _f32),
            pltpu.VMEM_SHARED((NPAD, 2 * DIM), _f32),
            pltpu.SemaphoreType.DMA,
            pltpu.SemaphoreType.DMA,
        ],
    )
    def k(msg_hbm, idx_hbm, zer_hbm, o0, o1, idx_v, msg_v, acc_sh, sem,
          sem2):
        c = lax.axis_index("c")
        s = lax.axis_index("s")
        wid = s * 2 + c
        pltpu.sync_copy(zer_hbm.at[pl.ds(s * RPT, RPT), pl.ds(0, 2 * DIM)],
                        acc_sh.at[pl.ds(s * RPT, RPT)])
        pltpu.sync_copy(idx_hbm.at[pl.ds(wid * CPW, CPW)], idx_v)
        plsc.subcore_barrier()

        def body(j, carry):
            cps = [
                pltpu.async_copy(
                    msg_hbm.at[pl.ds(wid * EPW + (j * GRP + b) * CHUNK, CHUNK),
                               pl.ds(0, 2 * DIM)],
                    msg_v.at[pl.ds(b * CHUNK, CHUNK)],
                    sem,
                )
                for b in range(GRP)
            ]
            for cp in cps:
                cp.wait()
            adds = [
                pltpu.async_copy(
                    msg_v.at[pl.ds(b * CHUNK, CHUNK)],
                    acc_sh.at[idx_v.at[j * GRP + b]], sem2, add=True)
                for b in range(GRP)
            ]
            for cp in adds:
                cp.wait()
            return carry

        lax.fori_loop(0, CPW // GRP, body, 0)
        plsc.subcore_barrier()

        @pl.when(c == 0)
        def _():
            pltpu.sync_copy(acc_sh.at[pl.ds(s * RPT, RPT)],
                            o0.at[pl.ds(s * RPT, RPT), pl.ds(0, 2 * DIM)])

        @pl.when(c == 1)
        def _():
            pltpu.sync_copy(acc_sh.at[pl.ds(s * RPT, RPT)],
                            o1.at[pl.ds(s * RPT, RPT), pl.ds(0, 2 * DIM)])

    return k(msg, idx2d, zeros_tab)


def _lin0_body(x_ref, w_ref, b_ref, o_ref):
    acc = jnp.dot(x_ref[...], w_ref[...], preferred_element_type=_f32)
    h = jnp.maximum(acc + b_ref[...], 0.0)
    o_ref[...] = jnp.concatenate(
        [h, jnp.zeros((NPAD, LW - DIM), _f32)], axis=1)


def _lin0(x, w_t, b_row):
    return pl.pallas_call(
        _lin0_body,
        out_shape=jax.ShapeDtypeStruct((NPAD, LW), _f32),
    )(x, w_t, b_row)


_TE = 1280


def _msg_body(ea_ref, os_ref, w1t_ref, b1_ref, w2t_ref, r_ref, o_ref):
    h1 = jnp.dot(ea_ref[...], w1t_ref[...], preferred_element_type=_f32)
    h1 = jnp.maximum(h1 + b1_ref[...], 0.0).astype(_bf16)
    w = jnp.dot(h1, w2t_ref[...], preferred_element_type=_f32)
    osrc = os_ref[...][:, 0:DIM]
    os_hi = osrc.astype(_bf16)
    os_lo = (osrc - os_hi.astype(_f32)).astype(_bf16)
    oshl = jnp.concatenate([os_hi, os_lo], axis=1)
    ose = jnp.dot(oshl, r_ref[...], preferred_element_type=_f32)
    q = ose[:, 0:128] * w[:, 0:128]
    for a in range(1, 8):
        q = q + ose[:, a * 128:(a + 1) * 128] * w[:, a * 128:(a + 1) * 128]
    msg = q[:, 0:32] + q[:, 32:64] + q[:, 64:96] + q[:, 96:128]
    msg = msg + ose[:, 1024:1024 + DIM]
    o_ref[...] = jnp.concatenate(
        [msg, jnp.ones((_TE, DIM), _f32), jnp.zeros((_TE, 2 * DIM), _f32)],
        axis=1)


def _msg(eap, osrc, w1t, b1, w2t, rmat):
    grid = (EP // _TE,)
    full = lambda shape: pl.BlockSpec(shape, lambda i: (0, 0))
    return pl.pallas_call(
        _msg_body,
        grid=grid,
        in_specs=[
            pl.BlockSpec((_TE, 4), lambda i: (i, 0)),
            pl.BlockSpec((_TE, LW), lambda i: (i, 0)),
            full((4, NUM_FEAT)),
            full((1, NUM_FEAT)),
            full((NUM_FEAT, DIM * DIM)),
            full((2 * DIM, DIM * DIM + DIM)),
        ],
        out_specs=pl.BlockSpec((_TE, LW), lambda i: (i, 0)),
        out_shape=jax.ShapeDtypeStruct((EP, LW), _f32),
    )(eap, osrc, w1t, b1, w2t, rmat)


def _sigm(x):
    return 1.0 / (1.0 + jnp.exp(-x))


_TN = 632


def _gru_body(p0_ref, p1_ref, h_ref, rw_ref, cb_ref,
              wih_ref, whh_ref, bih_ref, bhh_ref, o_ref):
    h = h_ref[...][:, 0:DIM]
    p0 = p0_ref[...]
    p1 = p1_ref[...]
    cnt = jnp.maximum(p0[:, DIM:2 * DIM] + p1[:, DIM:2 * DIM], 1.0)
    agg = (p0[:, 0:DIM] + p1[:, 0:DIM]) / cnt
    m = agg + jnp.dot(h, rw_ref[...], preferred_element_type=_f32) + cb_ref[...]
    m = jnp.maximum(m, 0.0)
    gi = jnp.dot(m, wih_ref[...], preferred_element_type=_f32) + bih_ref[...]
    gh = jnp.dot(h, whh_ref[...], preferred_element_type=_f32) + bhh_ref[...]
    r = _sigm(gi[:, 0:DIM] + gh[:, 0:DIM])
    z = _sigm(gi[:, DIM:2 * DIM] + gh[:, DIM:2 * DIM])
    n = jnp.tanh(gi[:, 2 * DIM:3 * DIM] + r * gh[:, 2 * DIM:3 * DIM])
    hn = (1.0 - z) * n + z * h
    o_ref[...] = jnp.concatenate(
        [hn, jnp.zeros((_TN, LW - DIM), _f32)], axis=1)


def _gru(p0, p1, h, root_w, conv_b_row, wih_t, whh_t, bih_row, bhh_row):
    grid = (NPAD // _TN,)
    row = lambda: pl.BlockSpec((_TN, LW), lambda i: (i, 0))
    full = lambda shape: pl.BlockSpec(shape, lambda i: (0, 0))
    return pl.pallas_call(
        _gru_body,
        grid=grid,
        in_specs=[
            row(), row(), row(),
            full((DIM, DIM)),
            full((1, DIM)),
            full((DIM, 3 * DIM)),
            full((DIM, 3 * DIM)),
            full((1, 3 * DIM)),
            full((1, 3 * DIM)),
        ],
        out_specs=pl.BlockSpec((_TN, LW), lambda i: (i, 0)),
        out_shape=jax.ShapeDtypeStruct((NPAD, LW), _f32),
    )(p0, p1, h, root_w, conv_b_row, wih_t, whh_t, bih_row, bhh_row)


def _s2s_body(out_ref, bcol_ref, brow_ref, wih_ref, whh_ref, bsum_ref,
              l1_ref, l1b_ref, l2_ref, l2b_ref, g_ref):
    out = out_ref[...][:, 0:DIM]
    bcol = bcol_ref[...]
    brow = brow_ref[...]
    gid_row = lax.broadcasted_iota(_i32, (1, NGRAPH), 1)
    gid_col = lax.broadcasted_iota(_i32, (NGRAPH, 1), 0)
    maskb = bcol == gid_row
    mask = maskb.astype(_f32)
    mask_t = (brow == gid_col).astype(_f32)
    valid = (bcol < NGRAPH).astype(_f32)

    q_star = jnp.zeros((NGRAPH, 2 * DIM), _f32)
    hh = jnp.zeros((NGRAPH, DIM), _f32)
    cc = jnp.zeros((NGRAPH, DIM), _f32)
    for _ in range(3):
        gates = (jnp.dot(q_star, wih_ref[...], preferred_element_type=_f32)
                 + jnp.dot(hh, whh_ref[...], preferred_element_type=_f32)
                 + bsum_ref[...])
        ig = _sigm(gates[:, 0:DIM])
        fg = _sigm(gates[:, DIM:2 * DIM])
        gg = jnp.tanh(gates[:, 2 * DIM:3 * DIM])
        og = _sigm(gates[:, 3 * DIM:4 * DIM])
        cc = fg * cc + ig * gg
        hh = og * jnp.tanh(cc)
        qn = jnp.dot(mask, hh, preferred_element_type=_f32)
        e = jnp.sum(out * qn, axis=1, keepdims=True)
        ems = jnp.max(jnp.where(maskb, e, -1e30), axis=0, keepdims=True)
        ems = jnp.where(ems < -1e29, 0.0, ems)
        emax_pn = jnp.sum(mask * ems, axis=1, keepdims=True)
        anum = jnp.exp(e - emax_pn) * valid
        aden_row = jnp.sum(mask * anum, axis=0, keepdims=True)
        aden_pn = jnp.sum(mask * aden_row, axis=1, keepdims=True)
        a = anum / (aden_pn + 1e-16)
        rr = jnp.dot(mask_t, a * out, preferred_element_type=_f32)
        q_star = jnp.concatenate([hh, rr], axis=1)
    g1 = jnp.dot(q_star, l1_ref[...], preferred_element_type=_f32) + l1b_ref[...]
    g1 = jnp.maximum(g1, 0.0)
    g_ref[...] = jnp.dot(g1, l2_ref[...], preferred_element_type=_f32) + l2b_ref[...]


def _s2s(out, bcol, brow, wih_t, whh_t, bsum, l1_t, l1b, l2_t, l2b):
    return pl.pallas_call(
        _s2s_body,
        out_shape=jax.ShapeDtypeStruct((NGRAPH, 1), _f32),
    )(out, bcol, brow, wih_t, whh_t, bsum, l1_t, l1b, l2_t, l2b)


def kernel(x, edge_index, edge_attr, batch, lin0_W, lin0_b, nn1_W, nn1_b,
           nn2_W, nn2_b, root_W, conv_b, gru_wih, gru_whh, gru_bih, gru_bhh,
           lstm_wih, lstm_whh, lstm_bih, lstm_bhh, lin1_W, lin1_b,
           lin2_W, lin2_b):
    src = edge_index[0].astype(_i32)
    dst = edge_index[1].astype(_i32)
    npad = EP - E
    src_p = jnp.concatenate([src, jnp.zeros((npad,), _i32)])
    dst_p = jnp.concatenate(
        [dst, N + (jnp.arange(npad, dtype=_i32) % 16)])
    ea_p = jnp.concatenate([edge_attr, jnp.zeros((npad, 4), _f32)])
    src2d = src_p.reshape(EP // CHUNK, CHUNK)
    dst2d = dst_p.reshape(EP // CHUNK, CHUNK)
    zeros_tab = jnp.zeros((NPAD, LW), _f32)
    x_p = jnp.concatenate([x, jnp.zeros((NPAD - N, NUM_FEAT), _f32)])
    batch_p = jnp.concatenate(
        [batch.astype(_i32), jnp.full((NPAD - N,), NGRAPH, _i32)])

    w1t = nn1_W.T
    b1 = nn1_b.reshape(1, NUM_FEAT)
    w2t = nn2_W.T.astype(_bf16)
    rkron = jnp.kron(jnp.eye(DIM, dtype=_f32), jnp.ones((1, DIM), _f32))
    bmat = nn2_b.reshape(DIM, DIM)
    rblk = jnp.concatenate([rkron, bmat], axis=1)
    rmat = jnp.concatenate([rblk, rblk], axis=0).astype(_bf16)
    conv_b_row = conv_b.reshape(1, DIM)
    wih_t = gru_wih.T
    whh_t = gru_whh.T
    bih_row = gru_bih.reshape(1, 3 * DIM)
    bhh_row = gru_bhh.reshape(1, 3 * DIM)

    out = _lin0(x_p, lin0_W.T, lin0_b.reshape(1, DIM))

    for _ in range(3):
        osrc = _sc_gather(out, src2d)
        msg = _msg(ea_p, osrc, w1t, b1, w2t, rmat)
        p0, p1 = _sc_scatter(msg, dst2d, zeros_tab)
        out = _gru(p0, p1, out, root_W, conv_b_row,
                   wih_t, whh_t, bih_row, bhh_row)

    g = _s2s(out,
             batch_p.reshape(NPAD, 1),
             batch_p.reshape(1, NPAD),
             lstm_wih.T, lstm_whh.T,
             (lstm_bih + lstm_bhh).reshape(1, 4 * DIM),
             lin1_W.T, lin1_b.reshape(1, DIM),
             lin2_W.T, lin2_b.reshape(1, 1))
    return (g.reshape(-1), out[:N, 0:DIM])


# speedup vs baseline: 1.2246x; 1.2246x over previous
import functools

import jax
import jax.numpy as jnp
from jax import lax
from jax.experimental import pallas as pl
from jax.experimental.pallas import tpu as pltpu
from jax.experimental.pallas import tpu_sc as plsc

N = 10000
E = 160000
NUM_FEAT = 128
DIM = 32
NGRAPH = 64

NW = 32
CHUNK = 128
CPW = 40
EP = NW * CPW * CHUNK
NPAD = N + 16
RPT = NPAD // 16
GRP = 8
EPW = CPW * CHUNK

_SC_PARAMS = pltpu.CompilerParams(use_tc_tiling_on_sc=False)

_f32 = jnp.float32
_bf16 = jnp.bfloat16
_i32 = jnp.int32


def _sc_gather(table, idx2d):
    mesh = plsc.VectorSubcoreMesh(core_axis_name="c", subcore_axis_name="s")

    @functools.partial(
        pl.kernel, mesh=mesh, compiler_params=_SC_PARAMS,
        out_type=jax.ShapeDtypeStruct((EP, DIM), _f32),
        scratch_types=[
            pltpu.VMEM((CPW, CHUNK), _i32),
            pltpu.VMEM((GRP * CHUNK, DIM), _f32),
            pltpu.SemaphoreType.DMA,
        ],
    )
    def k(table_hbm, idx_hbm, out_hbm, idx_v, rows_v, sem):
        c = lax.axis_index("c")
        s = lax.axis_index("s")
        wid = s * 2 + c
        pltpu.sync_copy(idx_hbm.at[pl.ds(wid * CPW, CPW)], idx_v)

        def body(j, carry):
            cps = [
                pltpu.async_copy(
                    table_hbm.at[idx_v.at[j * GRP + b]],
                    rows_v.at[pl.ds(b * CHUNK, CHUNK)],
                    sem,
                )
                for b in range(GRP)
            ]
            for cp in cps:
                cp.wait()
            pltpu.sync_copy(
                rows_v,
                out_hbm.at[pl.ds(wid * EPW + j * GRP * CHUNK, GRP * CHUNK)],
            )
            return carry

        lax.fori_loop(0, CPW // GRP, body, 0)

    return k(table, idx2d)


def _sc_scatter_cnt(msg, idx2d, zeros_tab, ones_sm):
    mesh = plsc.VectorSubcoreMesh(core_axis_name="c", subcore_axis_name="s")
    ot = jax.ShapeDtypeStruct((NPAD, DIM), _f32)

    @functools.partial(
        pl.kernel, mesh=mesh, compiler_params=_SC_PARAMS,
        out_type=(ot, ot, ot, ot),
        scratch_types=[
            pltpu.VMEM((CPW, CHUNK), _i32),
            pltpu.VMEM((GRP * CHUNK, DIM), _f32),
            pltpu.VMEM((CHUNK, DIM), _f32),
            pltpu.VMEM_SHARED((NPAD, DIM), _f32),
            pltpu.VMEM_SHARED((NPAD, DIM), _f32),
            pltpu.SemaphoreType.DMA,
            pltpu.SemaphoreType.DMA,
        ],
    )
    def k(msg_hbm, idx_hbm, zer_hbm, one_hbm, o0, o1, n0, n1,
          idx_v, msg_v, ones_v, acc_sh, cnt_sh, sem, sem2):
        c = lax.axis_index("c")
        s = lax.axis_index("s")
        wid = s * 2 + c
        pltpu.sync_copy(zer_hbm.at[pl.ds(s * RPT, RPT)],
                        acc_sh.at[pl.ds(s * RPT, RPT)])
        pltpu.sync_copy(zer_hbm.at[pl.ds(s * RPT, RPT)],
                        cnt_sh.at[pl.ds(s * RPT, RPT)])
        pltpu.sync_copy(one_hbm, ones_v)
        pltpu.sync_copy(idx_hbm.at[pl.ds(wid * CPW, CPW)], idx_v)
        plsc.subcore_barrier()

        def body(j, carry):
            cps = [
                pltpu.async_copy(
                    msg_hbm.at[pl.ds(wid * EPW + (j * GRP + b) * CHUNK, CHUNK)],
                    msg_v.at[pl.ds(b * CHUNK, CHUNK)],
                    sem,
                )
                for b in range(GRP)
            ]
            for cp in cps:
                cp.wait()
            adds = []
            for b in range(GRP):
                adds.append(pltpu.async_copy(
                    msg_v.at[pl.ds(b * CHUNK, CHUNK)],
                    acc_sh.at[idx_v.at[j * GRP + b]], sem2, add=True))
                adds.append(pltpu.async_copy(
                    ones_v, cnt_sh.at[idx_v.at[j * GRP + b]], sem2, add=True))
            for cp in adds:
                cp.wait()
            return carry

        lax.fori_loop(0, CPW // GRP, body, 0)
        plsc.subcore_barrier()

        @pl.when(c == 0)
        def _():
            pltpu.sync_copy(acc_sh.at[pl.ds(s * RPT, RPT)],
                            o0.at[pl.ds(s * RPT, RPT)])
            pltpu.sync_copy(cnt_sh.at[pl.ds(s * RPT, RPT)],
                            n0.at[pl.ds(s * RPT, RPT)])

        @pl.when(c == 1)
        def _():
            pltpu.sync_copy(acc_sh.at[pl.ds(s * RPT, RPT)],
                            o1.at[pl.ds(s * RPT, RPT)])
            pltpu.sync_copy(cnt_sh.at[pl.ds(s * RPT, RPT)],
                            n1.at[pl.ds(s * RPT, RPT)])

    return k(msg, idx2d, zeros_tab, ones_sm)


def _sc_scatter(msg, idx2d, zeros_tab):
    mesh = plsc.VectorSubcoreMesh(core_axis_name="c", subcore_axis_name="s")
    ot = jax.ShapeDtypeStruct((NPAD, DIM), _f32)

    @functools.partial(
        pl.kernel, mesh=mesh, compiler_params=_SC_PARAMS,
        out_type=(ot, ot),
        scratch_types=[
            pltpu.VMEM((CPW, CHUNK), _i32),
            pltpu.VMEM((GRP * CHUNK, DIM), _f32),
            pltpu.VMEM_SHARED((NPAD, DIM), _f32),
            pltpu.SemaphoreType.DMA,
            pltpu.SemaphoreType.DMA,
        ],
    )
    def k(msg_hbm, idx_hbm, zer_hbm, o0, o1, idx_v, msg_v, acc_sh, sem,
          sem2):
        c = lax.axis_index("c")
        s = lax.axis_index("s")
        wid = s * 2 + c
        pltpu.sync_copy(zer_hbm.at[pl.ds(s * RPT, RPT)],
                        acc_sh.at[pl.ds(s * RPT, RPT)])
        pltpu.sync_copy(idx_hbm.at[pl.ds(wid * CPW, CPW)], idx_v)
        plsc.subcore_barrier()

        def body(j, carry):
            cps = [
                pltpu.async_copy(
                    msg_hbm.at[pl.ds(wid * EPW + (j * GRP + b) * CHUNK, CHUNK)],
                    msg_v.at[pl.ds(b * CHUNK, CHUNK)],
                    sem,
                )
                for b in range(GRP)
            ]
            for cp in cps:
                cp.wait()
            adds = [
                pltpu.async_copy(msg_v.at[pl.ds(b * CHUNK, CHUNK)],
                                 acc_sh.at[idx_v.at[j * GRP + b]], sem2,
                                 add=True)
                for b in range(GRP)
            ]
            for cp in adds:
                cp.wait()
            return carry

        lax.fori_loop(0, CPW // GRP, body, 0)
        plsc.subcore_barrier()

        @pl.when(c == 0)
        def _():
            pltpu.sync_copy(acc_sh.at[pl.ds(s * RPT, RPT)],
                            o0.at[pl.ds(s * RPT, RPT)])

        @pl.when(c == 1)
        def _():
            pltpu.sync_copy(acc_sh.at[pl.ds(s * RPT, RPT)],
                            o1.at[pl.ds(s * RPT, RPT)])

    return k(msg, idx2d, zeros_tab)


def _lin0_body(x_ref, w_ref, b_ref, o_ref):
    acc = jnp.dot(x_ref[...], w_ref[...], preferred_element_type=_f32)
    o_ref[...] = jnp.maximum(acc + b_ref[...], 0.0)


def _lin0(x, w_t, b_row):
    return pl.pallas_call(
        _lin0_body,
        out_shape=jax.ShapeDtypeStruct((N, DIM), _f32),
    )(x, w_t, b_row)


_TE = 1280


def _msg_body(ea_ref, os_ref, w1t_ref, b1_ref, w2t_ref, r_ref, o_ref):
    h1 = jnp.dot(ea_ref[...], w1t_ref[...], preferred_element_type=_f32)
    h1 = jnp.maximum(h1 + b1_ref[...], 0.0).astype(_bf16)
    w = jnp.dot(h1, w2t_ref[...], preferred_element_type=_f32)
    osrc = os_ref[...]
    os_hi = osrc.astype(_bf16)
    os_lo = (osrc - os_hi.astype(_f32)).astype(_bf16)
    oshl = jnp.concatenate([os_hi, os_lo], axis=1)
    ose = jnp.dot(oshl, r_ref[...], preferred_element_type=_f32)
    q = ose[:, 0:128] * w[:, 0:128]
    for a in range(1, 8):
        q = q + ose[:, a * 128:(a + 1) * 128] * w[:, a * 128:(a + 1) * 128]
    msg = q[:, 0:32] + q[:, 32:64] + q[:, 64:96] + q[:, 96:128]
    msg = msg + ose[:, 1024:1024 + DIM]
    o_ref[...] = msg


def _msg(eap, osrc, w1t, b1, w2t, rmat):
    grid = (E // _TE,)
    full = lambda shape: pl.BlockSpec(shape, lambda i: (0, 0))
    return pl.pallas_call(
        _msg_body,
        grid=grid,
        in_specs=[
            pl.BlockSpec((_TE, 4), lambda i: (i, 0)),
            pl.BlockSpec((_TE, DIM), lambda i: (i, 0)),
            full((4, NUM_FEAT)),
            full((1, NUM_FEAT)),
            full((NUM_FEAT, DIM * DIM)),
            full((2 * DIM, DIM * DIM + DIM)),
        ],
        out_specs=pl.BlockSpec((_TE, DIM), lambda i: (i, 0)),
        out_shape=jax.ShapeDtypeStruct((EP, DIM), _f32),
    )(eap, osrc, w1t, b1, w2t, rmat)


def _sigm(x):
    return 1.0 / (1.0 + jnp.exp(-x))


_TN = 1000


def _gru_body(p0_ref, p1_ref, c0_ref, c1_ref, h_ref, rw_ref, cb_ref,
              wih_ref, whh_ref, bih_ref, bhh_ref, o_ref):
    h = h_ref[...]
    cnt = jnp.maximum(c0_ref[...] + c1_ref[...], 1.0)
    agg = (p0_ref[...] + p1_ref[...]) / cnt
    m = agg + jnp.dot(h, rw_ref[...], preferred_element_type=_f32) + cb_ref[...]
    m = jnp.maximum(m, 0.0)
    gi = jnp.dot(m, wih_ref[...], preferred_element_type=_f32) + bih_ref[...]
    gh = jnp.dot(h, whh_ref[...], preferred_element_type=_f32) + bhh_ref[...]
    r = _sigm(gi[:, 0:DIM] + gh[:, 0:DIM])
    z = _sigm(gi[:, DIM:2 * DIM] + gh[:, DIM:2 * DIM])
    n = jnp.tanh(gi[:, 2 * DIM:3 * DIM] + r * gh[:, 2 * DIM:3 * DIM])
    o_ref[...] = (1.0 - z) * n + z * h


def _gru(p0, p1, c0, c1, h, root_w, conv_b_row, wih_t, whh_t, bih_row, bhh_row):
    grid = (N // _TN,)
    row = lambda: pl.BlockSpec((_TN, DIM), lambda i: (i, 0))
    full = lambda shape: pl.BlockSpec(shape, lambda i: (0, 0))
    return pl.pallas_call(
        _gru_body,
        grid=grid,
        in_specs=[
            row(), row(), row(), row(), row(),
            full((DIM, DIM)),
            full((1, DIM)),
            full((DIM, 3 * DIM)),
            full((DIM, 3 * DIM)),
            full((1, 3 * DIM)),
            full((1, 3 * DIM)),
        ],
        out_specs=pl.BlockSpec((_TN, DIM), lambda i: (i, 0)),
        out_shape=jax.ShapeDtypeStruct((N, DIM), _f32),
    )(p0, p1, c0, c1, h, root_w, conv_b_row, wih_t, whh_t, bih_row, bhh_row)


def _s2s_body(out_ref, bcol_ref, brow_ref, wih_ref, whh_ref, bsum_ref,
              l1_ref, l1b_ref, l2_ref, l2b_ref, g_ref):
    out = out_ref[...]
    bcol = bcol_ref[...]
    brow = brow_ref[...]
    gid_row = lax.broadcasted_iota(_i32, (1, NGRAPH), 1)
    gid_col = lax.broadcasted_iota(_i32, (NGRAPH, 1), 0)
    maskb = bcol == gid_row
    mask = maskb.astype(_f32)
    mask_t = (brow == gid_col).astype(_f32)

    q_star = jnp.zeros((NGRAPH, 2 * DIM), _f32)
    hh = jnp.zeros((NGRAPH, DIM), _f32)
    cc = jnp.zeros((NGRAPH, DIM), _f32)
    for _ in range(3):
        gates = (jnp.dot(q_star, wih_ref[...], preferred_element_type=_f32)
                 + jnp.dot(hh, whh_ref[...], preferred_element_type=_f32)
                 + bsum_ref[...])
        ig = _sigm(gates[:, 0:DIM])
        fg = _sigm(gates[:, DIM:2 * DIM])
        gg = jnp.tanh(gates[:, 2 * DIM:3 * DIM])
        og = _sigm(gates[:, 3 * DIM:4 * DIM])
        cc = fg * cc + ig * gg
        hh = og * jnp.tanh(cc)
        qn = jnp.dot(mask, hh, preferred_element_type=_f32)
        e = jnp.sum(out * qn, axis=1, keepdims=True)
        ems = jnp.max(jnp.where(maskb, e, -1e30), axis=0, keepdims=True)
        ems = jnp.where(ems < -1e29, 0.0, ems)
        emax_pn = jnp.sum(mask * ems, axis=1, keepdims=True)
        anum = jnp.exp(e - emax_pn)
        aden_row = jnp.sum(mask * anum, axis=0, keepdims=True)
        aden_pn = jnp.sum(mask * aden_row, axis=1, keepdims=True)
        a = anum / (aden_pn + 1e-16)
        rr = jnp.dot(mask_t, a * out, preferred_element_type=_f32)
        q_star = jnp.concatenate([hh, rr], axis=1)
    g1 = jnp.dot(q_star, l1_ref[...], preferred_element_type=_f32) + l1b_ref[...]
    g1 = jnp.maximum(g1, 0.0)
    g_ref[...] = jnp.dot(g1, l2_ref[...], preferred_element_type=_f32) + l2b_ref[...]


def _s2s(out, bcol, brow, wih_t, whh_t, bsum, l1_t, l1b, l2_t, l2b):
    return pl.pallas_call(
        _s2s_body,
        out_shape=jax.ShapeDtypeStruct((NGRAPH, 1), _f32),
    )(out, bcol, brow, wih_t, whh_t, bsum, l1_t, l1b, l2_t, l2b)


def kernel(x, edge_index, edge_attr, batch, lin0_W, lin0_b, nn1_W, nn1_b,
           nn2_W, nn2_b, root_W, conv_b, gru_wih, gru_whh, gru_bih, gru_bhh,
           lstm_wih, lstm_whh, lstm_bih, lstm_bhh, lin1_W, lin1_b,
           lin2_W, lin2_b):
    src = edge_index[0].astype(_i32)
    dst = edge_index[1].astype(_i32)
    npad = EP - E
    src_p = jnp.concatenate([src, jnp.zeros((npad,), _i32)])
    dst_p = jnp.concatenate(
        [dst, N + (jnp.arange(npad, dtype=_i32) % 16)])
    src2d = src_p.reshape(EP // CHUNK, CHUNK)
    dst2d = dst_p.reshape(EP // CHUNK, CHUNK)
    zeros_tab = jnp.zeros((NPAD, DIM), _f32)
    ones_sm = jnp.ones((CHUNK, DIM), _f32)

    w1t = nn1_W.T
    b1 = nn1_b.reshape(1, NUM_FEAT)
    w2t = nn2_W.T.astype(_bf16)
    rkron = jnp.kron(jnp.eye(DIM, dtype=_f32), jnp.ones((1, DIM), _f32))
    bmat = nn2_b.reshape(DIM, DIM)
    rblk = jnp.concatenate([rkron, bmat], axis=1)
    rmat = jnp.concatenate([rblk, rblk], axis=0).astype(_bf16)
    conv_b_row = conv_b.reshape(1, DIM)
    wih_t = gru_wih.T
    whh_t = gru_whh.T
    bih_row = gru_bih.reshape(1, 3 * DIM)
    bhh_row = gru_bhh.reshape(1, 3 * DIM)

    out = _lin0(x, lin0_W.T, lin0_b.reshape(1, DIM))

    c0s = c1s = None
    for it in range(3):
        osrc = _sc_gather(out, src2d)
        msg = _msg(edge_attr, osrc, w1t, b1, w2t, rmat)
        if it == 0:
            p0, p1, n0, n1 = _sc_scatter_cnt(msg, dst2d, zeros_tab, ones_sm)
            c0s, c1s = n0[:N], n1[:N]
        else:
            p0, p1 = _sc_scatter(msg, dst2d, zeros_tab)
        out = _gru(p0[:N], p1[:N], c0s, c1s, out, root_W, conv_b_row,
                   wih_t, whh_t, bih_row, bhh_row)

    g = _s2s(out,
             batch.astype(_i32).reshape(N, 1),
             batch.astype(_i32).reshape(1, N),
             lstm_wih.T, lstm_whh.T,
             (lstm_bih + lstm_bhh).reshape(1, 4 * DIM),
             lin1_W.T, lin1_b.reshape(1, DIM),
             lin2_W.T, lin2_b.reshape(1, 1))
    return (g.reshape(-1), out)
